# baseline (speedup 1.0000x reference)
import functools

import jax
import jax.numpy as jnp
import numpy as np
from jax import lax
from jax.experimental import pallas as pl
from jax.experimental.pallas import tpu as pltpu
from jax.experimental.pallas import tpu_sc as plsc

_NC = 2
_NS = 16
_NW = _NC * _NS
_D = 128
_DH = _D // 2
_B = 4 * 8192
_BP = _B // _NW
_C = 128
_NCH = _BP // _C
_SC = 2
_CS = _C * _SC
_NSC = _NCH // _SC
_RING = 3

_INVF_TURNS = np.concatenate(
    [0.5 / (2.0 * np.pi) / (10000.0 ** (np.arange(0, _D, 2) / _D))] * 2)
_SIN_POLY = (6.28318347, -41.34148026, 81.59765525, -76.59489967,
             41.26979637, -12.37227203)


@functools.partial(
    pl.kernel,
    mesh=plsc.VectorSubcoreMesh(core_axis_name="c", subcore_axis_name="s"),
    out_type=jax.ShapeDtypeStruct((_B, _D), jnp.float32),
    scratch_types=(
        [pltpu.VMEM((_NCH, _C), jnp.int32)]
        + [pltpu.VMEM((_CS, _DH), jnp.float32) for _ in range(_RING)]
        + [pltpu.SemaphoreType.DMA for _ in range(2 * _RING)]
    ),
    compiler_params=pltpu.CompilerParams(use_tc_tiling_on_sc=False),
)
def _rope_gather_cos(pos_hbm, cos_hbm, cos_out, idx_v, *rest):
    bufs = rest[0:_RING]
    sg = rest[_RING:2 * _RING]
    sw = rest[2 * _RING:3 * _RING]
    wid = lax.axis_index("s") * _NC + lax.axis_index("c")
    base = wid * _BP
    gh, wh = {}, {}

    def issue_gathers(s):
        b = s % _RING
        return tuple(
            pltpu.async_copy(cos_hbm.at[idx_v.at[s * _SC + j]],
                             bufs[b].at[pl.ds(j * _C, _C)], sg[b])
            for j in range(_SC)
        )

    def issue_writes(s):
        b = s % _RING
        off = base + s * _CS
        return tuple(
            pltpu.async_copy(bufs[b], cos_out.at[pl.ds(off, _CS), pl.ds(col, _DH)],
                             sw[b])
            for col in (0, _DH)
        )

    pltpu.sync_copy(pos_hbm.at[wid, pl.ds(0, _SC)], idx_v.at[pl.ds(0, _SC)])
    gh[0] = issue_gathers(0)
    pltpu.sync_copy(pos_hbm.at[wid, pl.ds(_SC, _NCH - _SC)],
                    idx_v.at[pl.ds(_SC, _NCH - _SC)])
    if _NSC > 1:
        gh[1] = issue_gathers(1)
    waited_w = set()
    for s in range(_NSC):
        for h in gh[s]:
            h.wait()
        wh[s] = issue_writes(s)
        if s + 2 < _NSC:
            p = s + 2 - _RING
            if p >= 0:
                for h in wh[p]:
                    h.wait()
                waited_w.add(p)
            gh[s + 2] = issue_gathers(s + 2)
    for s in range(_NSC):
        if s not in waited_w:
            for h in wh[s]:
                h.wait()


_RCH = 4
_TG = (_B // _C) // _RCH


def _sin_body(pos_ref, invf_ref, sin_ref):
    p2 = pos_ref[0].astype(jnp.float32)
    h = _RCH // 2
    pp = p2.reshape(h, 2, _D)
    pe = pp[:, 0, :].reshape(h, _D, 1)
    po = pp[:, 1, :].reshape(h, _D, 1)
    pack = jnp.concatenate(
        [jnp.broadcast_to(pe, (h, _D, _DH)), jnp.broadcast_to(po, (h, _D, _DH))],
        axis=2)
    x = pack * invf_ref[0].reshape(1, 1, _D)
    r = x - jnp.floor(x + 0.5)
    s2 = r * r
    q = jnp.float32(_SIN_POLY[-1])
    for ci in _SIN_POLY[-2::-1]:
        q = q * s2 + jnp.float32(ci)
    s = r * q
    se, so = s[:, :, :_DH], s[:, :, _DH:]
    for a in range(h):
        sin_ref[0, 2 * a] = jnp.concatenate([se[a], se[a]], axis=1)
        sin_ref[0, 2 * a + 1] = jnp.concatenate([so[a], so[a]], axis=1)


@jax.jit
def _hybrid(pos2, cos_h):
    cos = _rope_gather_cos(pos2, cos_h)
    invf = jnp.asarray(_INVF_TURNS, dtype=jnp.float32).reshape(1, _D)
    sin = pl.pallas_call(
        _sin_body,
        grid=(_TG,),
        in_specs=[
            pl.BlockSpec((1, _RCH, _C), lambda i: (i, 0, 0)),
            pl.BlockSpec((1, _D), lambda i: (0, 0)),
        ],
        out_specs=pl.BlockSpec((1, _RCH, _D, _D), lambda i: (i, 0, 0, 0)),
        out_shape=jax.ShapeDtypeStruct((_TG, _RCH, _D, _D), jnp.float32),
    )(pos2.reshape(_TG, _RCH, _C), invf)
    return cos, sin


def kernel(positions, cos_cached, sin_cached):
    shape = positions.shape
    n_rows, d = cos_cached.shape
    pos2 = (positions * 2).reshape(_NW, _NCH, _C)
    cos_h = cos_cached.reshape(n_rows * 2, d // 2)
    cos, sin = _hybrid(pos2, cos_h)
    return (cos.reshape(*shape, _D), sin.reshape(*shape, _D))


# speedup vs baseline: 1.5882x; 1.5882x over previous
import functools

import jax
import jax.numpy as jnp
import numpy as np
from jax import lax
from jax.experimental import pallas as pl
from jax.experimental.pallas import tpu as pltpu
from jax.experimental.pallas import tpu_sc as plsc

_NC = 2
_NS = 16
_NW = _NC * _NS
_D = 128
_DH = _D // 2
_B = 4 * 8192
_BP = _B // _NW
_C = 128
_NCH = _BP // _C
_SC = 2
_CS = _C * _SC
_NSC = _NCH // _SC
_RING = 3

_INVF_TURNS = np.concatenate(
    [0.5 / (2.0 * np.pi) / (10000.0 ** (np.arange(0, _D, 2) / _D))] * 2)
_SIN_POLY = (6.28318347, -41.34148026, 81.59765525, -76.59489967,
             41.26979637, -12.37227203)


@functools.partial(
    pl.kernel,
    mesh=plsc.VectorSubcoreMesh(core_axis_name="c", subcore_axis_name="s"),
    out_type=jax.ShapeDtypeStruct((_B, _D), jnp.float32),
    scratch_types=(
        [pltpu.VMEM((_NCH, _C), jnp.int32)]
        + [pltpu.VMEM((_CS, _DH), jnp.float32) for _ in range(_RING)]
        + [pltpu.SemaphoreType.DMA for _ in range(2 * _RING)]
    ),
    compiler_params=pltpu.CompilerParams(use_tc_tiling_on_sc=False),
)
def _rope_gather_cos(pos_hbm, cos_hbm, cos_out, idx_v, *rest):
    bufs = rest[0:_RING]
    sg = rest[_RING:2 * _RING]
    sw = rest[2 * _RING:3 * _RING]
    wid = lax.axis_index("s") * _NC + lax.axis_index("c")
    base = wid * _BP
    gh, wh = {}, {}

    def issue_gathers(s):
        b = s % _RING
        return tuple(
            pltpu.async_copy(cos_hbm.at[idx_v.at[s * _SC + j]],
                             bufs[b].at[pl.ds(j * _C, _C)], sg[b])
            for j in range(_SC)
        )

    def issue_writes(s):
        b = s % _RING
        off = base + s * _CS
        return tuple(
            pltpu.async_copy(bufs[b], cos_out.at[pl.ds(off, _CS), pl.ds(col, _DH)],
                             sw[b])
            for col in (0, _DH)
        )

    pltpu.sync_copy(pos_hbm.at[wid, pl.ds(0, _SC)], idx_v.at[pl.ds(0, _SC)])
    gh[0] = issue_gathers(0)
    pltpu.sync_copy(pos_hbm.at[wid, pl.ds(_SC, _NCH - _SC)],
                    idx_v.at[pl.ds(_SC, _NCH - _SC)])
    if _NSC > 1:
        gh[1] = issue_gathers(1)
    waited_w = set()
    for s in range(_NSC):
        for h in gh[s]:
            h.wait()
        wh[s] = issue_writes(s)
        if s + 2 < _NSC:
            p = s + 2 - _RING
            if p >= 0:
                for h in wh[p]:
                    h.wait()
                waited_w.add(p)
            gh[s + 2] = issue_gathers(s + 2)
    for s in range(_NSC):
        if s not in waited_w:
            for h in wh[s]:
                h.wait()


_RCH = 16
_TG = (_B // _C) // _RCH


def _sin_body(pos_ref, invf_ref, sin_ref):
    p2 = pos_ref[0].astype(jnp.float32)
    h = _RCH // 2
    pp = p2.reshape(h, 2, _D)
    pe = pp[:, 0, :].reshape(h, _D, 1)
    po = pp[:, 1, :].reshape(h, _D, 1)
    pack = jnp.concatenate(
        [jnp.broadcast_to(pe, (h, _D, _DH)), jnp.broadcast_to(po, (h, _D, _DH))],
        axis=2)
    x = pack * invf_ref[0].reshape(1, 1, _D)
    r = x - jnp.floor(x + 0.5)
    s2 = r * r
    q = jnp.float32(_SIN_POLY[-1])
    for ci in _SIN_POLY[-2::-1]:
        q = q * s2 + jnp.float32(ci)
    s = r * q
    se, so = s[:, :, :_DH], s[:, :, _DH:]
    for a in range(h):
        sin_ref[0, 2 * a] = jnp.concatenate([se[a], se[a]], axis=1)
        sin_ref[0, 2 * a + 1] = jnp.concatenate([so[a], so[a]], axis=1)


@jax.jit
def _hybrid(pos2, cos_h):
    cos = _rope_gather_cos(pos2, cos_h)
    invf = jnp.asarray(_INVF_TURNS, dtype=jnp.float32).reshape(1, _D)
    sin = pl.pallas_call(
        _sin_body,
        grid=(_TG,),
        in_specs=[
            pl.BlockSpec((1, _RCH, _C), lambda i: (i, 0, 0)),
            pl.BlockSpec((1, _D), lambda i: (0, 0)),
        ],
        out_specs=pl.BlockSpec((1, _RCH, _D, _D), lambda i: (i, 0, 0, 0)),
        out_shape=jax.ShapeDtypeStruct((_TG, _RCH, _D, _D), jnp.float32),
    )(pos2.reshape(_TG, _RCH, _C), invf)
    return cos, sin


def kernel(positions, cos_cached, sin_cached):
    shape = positions.shape
    n_rows, d = cos_cached.shape
    pos2 = (positions * 2).reshape(_NW, _NCH, _C)
    cos_h = cos_cached.reshape(n_rows * 2, d // 2)
    cos, sin = _hybrid(pos2, cos_h)
    return (cos.reshape(*shape, _D), sin.reshape(*shape, _D))


# speedup vs baseline: 1.6976x; 1.0689x over previous
import functools

import jax
import jax.numpy as jnp
import numpy as np
from jax import lax
from jax.experimental import pallas as pl
from jax.experimental.pallas import tpu as pltpu
from jax.experimental.pallas import tpu_sc as plsc

_NC = 2
_NS = 16
_NW = _NC * _NS
_D = 128
_DH = _D // 2
_B = 4 * 8192
_BP = _B // _NW
_C = 128
_NCH = _BP // _C
_SC = 2
_CS = _C * _SC
_NSC = _NCH // _SC
_RING = 3

_INVF_TURNS = np.concatenate(
    [0.5 / (2.0 * np.pi) / (10000.0 ** (np.arange(0, _D, 2) / _D))] * 2)
_SIN_POLY = (6.28318347, -41.34148026, 81.59765525, -76.59489967,
             41.26979637, -12.37227203)


@functools.partial(
    pl.kernel,
    mesh=plsc.VectorSubcoreMesh(core_axis_name="c", subcore_axis_name="s"),
    out_type=jax.ShapeDtypeStruct((_B, _D), jnp.float32),
    scratch_types=(
        [pltpu.VMEM((_NCH, _C), jnp.int32)]
        + [pltpu.VMEM((_CS, _DH), jnp.float32) for _ in range(_RING)]
        + [pltpu.SemaphoreType.DMA for _ in range(2 * _RING)]
    ),
    compiler_params=pltpu.CompilerParams(use_tc_tiling_on_sc=False),
)
def _rope_gather_cos(pos_hbm, cos_hbm, cos_out, idx_v, *rest):
    bufs = rest[0:_RING]
    sg = rest[_RING:2 * _RING]
    sw = rest[2 * _RING:3 * _RING]
    wid = lax.axis_index("s") * _NC + lax.axis_index("c")
    base = wid * _BP
    gh, wh = {}, {}

    def issue_gathers(s):
        b = s % _RING
        return tuple(
            pltpu.async_copy(cos_hbm.at[idx_v.at[s * _SC + j]],
                             bufs[b].at[pl.ds(j * _C, _C)], sg[b])
            for j in range(_SC)
        )

    def issue_writes(s):
        b = s % _RING
        off = base + s * _CS
        return tuple(
            pltpu.async_copy(bufs[b], cos_out.at[pl.ds(off, _CS), pl.ds(col, _DH)],
                             sw[b])
            for col in (0, _DH)
        )

    pltpu.sync_copy(pos_hbm.at[wid, pl.ds(0, _SC)], idx_v.at[pl.ds(0, _SC)])
    gh[0] = issue_gathers(0)
    pltpu.sync_copy(pos_hbm.at[wid, pl.ds(_SC, _NCH - _SC)],
                    idx_v.at[pl.ds(_SC, _NCH - _SC)])
    if _NSC > 1:
        gh[1] = issue_gathers(1)
    waited_w = set()
    for s in range(_NSC):
        for h in gh[s]:
            h.wait()
        wh[s] = issue_writes(s)
        if s + 2 < _NSC:
            p = s + 2 - _RING
            if p >= 0:
                for h in wh[p]:
                    h.wait()
                waited_w.add(p)
            gh[s + 2] = issue_gathers(s + 2)
    for s in range(_NSC):
        if s not in waited_w:
            for h in wh[s]:
                h.wait()


_RCH = 32
_TG = (_B // _C) // _RCH


def _sin_body(pos_ref, invf_ref, sin_ref):
    p2 = pos_ref[0].astype(jnp.float32)
    h = _RCH // 2
    pp = p2.reshape(h, 2, _D)
    pe = pp[:, 0, :].reshape(h, _D, 1)
    po = pp[:, 1, :].reshape(h, _D, 1)
    pack = jnp.concatenate(
        [jnp.broadcast_to(pe, (h, _D, _DH)), jnp.broadcast_to(po, (h, _D, _DH))],
        axis=2)
    x = pack * invf_ref[0].reshape(1, 1, _D)
    r = x - jnp.floor(x + 0.5)
    s2 = r * r
    q = jnp.float32(_SIN_POLY[-1])
    for ci in _SIN_POLY[-2::-1]:
        q = q * s2 + jnp.float32(ci)
    s = r * q
    se, so = s[:, :, :_DH], s[:, :, _DH:]
    for a in range(h):
        sin_ref[0, 2 * a] = jnp.concatenate([se[a], se[a]], axis=1)
        sin_ref[0, 2 * a + 1] = jnp.concatenate([so[a], so[a]], axis=1)


@jax.jit
def _hybrid(pos2, cos_h):
    cos = _rope_gather_cos(pos2, cos_h)
    invf = jnp.asarray(_INVF_TURNS, dtype=jnp.float32).reshape(1, _D)
    sin = pl.pallas_call(
        _sin_body,
        grid=(_TG,),
        in_specs=[
            pl.BlockSpec((1, _RCH, _C), lambda i: (i, 0, 0)),
            pl.BlockSpec((1, _D), lambda i: (0, 0)),
        ],
        out_specs=pl.BlockSpec((1, _RCH, _D, _D), lambda i: (i, 0, 0, 0)),
        out_shape=jax.ShapeDtypeStruct((_TG, _RCH, _D, _D), jnp.float32),
    )(pos2.reshape(_TG, _RCH, _C), invf)
    return cos, sin


def kernel(positions, cos_cached, sin_cached):
    shape = positions.shape
    n_rows, d = cos_cached.shape
    pos2 = (positions * 2).reshape(_NW, _NCH, _C)
    cos_h = cos_cached.reshape(n_rows * 2, d // 2)
    cos, sin = _hybrid(pos2, cos_h)
    return (cos.reshape(*shape, _D), sin.reshape(*shape, _D))
